# baseline (speedup 1.0000x reference)
import functools

import jax
import jax.numpy as jnp
from jax import lax
from jax.experimental import pallas as pl
from jax.experimental.pallas import tpu as pltpu
from jax.experimental.pallas import tpu_sc as plsc

_NC, _NS, _LANES = 2, 16, 16
_NW = _NC * _NS

_LVL_BITS = (8, 8, 8, 7)
_LVL_SHIFTS = (23, 15, 7, 0)


def _sc_select_body(k, nvec, x_hbm, thr_hbm, xbuf, hist, tvec):
    lane = lax.iota(jnp.int32, _LANES)
    ones = jnp.full((_LANES,), 1, jnp.int32)
    zeros16 = jnp.zeros((_LANES,), jnp.int32)
    wid = lax.axis_index("c") * _NS + lax.axis_index("s")

    rows_per_worker = 2
    acc_thr = zeros16
    for j in range(rows_per_worker):
        r = wid * rows_per_worker + j
        pltpu.sync_copy(x_hbm.at[r], xbuf)
        kk = jnp.int32(k)
        prefix = jnp.int32(0)
        for lvl in range(4):
            shift = _LVL_SHIFTS[lvl]
            nbits = _LVL_BITS[lvl]
            nb = 1 << nbits

            def zbody(b, _):
                hist[pl.ds(b * _LANES, _LANES)] = zeros16
                return jnp.int32(0)

            lax.fori_loop(0, nb, zbody, jnp.int32(0))

            pfx = prefix

            def hbody(i, _, lvl=lvl, shift=shift, nbits=nbits, nb=nb, pfx=pfx):
                v = xbuf[pl.ds(i * _LANES, _LANES)]
                if lvl == 0:
                    v = v & jnp.int32(0x7FFFFFFF)
                    xbuf[pl.ds(i * _LANES, _LANES)] = v
                b = (v >> shift) & jnp.int32(nb - 1)
                idx = b * _LANES + lane
                if lvl == 0:
                    plsc.addupdate_scatter(hist, [idx], ones)
                else:
                    sel = (v >> (shift + nbits)) == pfx
                    plsc.addupdate_scatter(hist, [idx], ones, mask=sel)
                return jnp.int32(0)

            lax.fori_loop(0, nvec, hbody, jnp.int32(0))

            def sbody(t, acc, nb=nb):
                bb = nb - 1 - t
                acc = acc + hist[pl.ds(bb * _LANES, _LANES)]
                hist[pl.ds(bb * _LANES, _LANES)] = acc
                return acc

            lax.fori_loop(0, nb, sbody, zeros16)

            lo = jnp.int32(0)
            step = nb >> 1
            while step >= 1:
                cand = lo + jnp.int32(step)
                candc = jnp.minimum(cand, nb - 1)
                sv = jnp.sum(hist[pl.ds(candc * _LANES, _LANES)])
                take = (cand <= nb - 1) & (sv >= kk)
                lo = jnp.where(take, cand, lo)
                step >>= 1
            upc = jnp.minimum(lo + 1, nb - 1)
            sv1 = jnp.sum(hist[pl.ds(upc * _LANES, _LANES)])
            above = jnp.where(lo + 1 <= nb - 1, sv1, jnp.int32(0))
            kk = kk - above
            prefix = (prefix << nbits) | lo
        acc_thr = jnp.where(lane == j, prefix, acc_thr)
    tvec[...] = acc_thr
    pltpu.sync_copy(tvec, thr_hbm.at[wid])


def _sc_select(xi, k):
    B, H = xi.shape
    mesh = plsc.VectorSubcoreMesh(
        core_axis_name="c", subcore_axis_name="s", num_cores=_NC,
        num_subcores=_NS)
    body = functools.partial(_sc_select_body, k, H // _LANES)
    return pl.kernel(
        body,
        out_type=jax.ShapeDtypeStruct((_NW, _LANES), jnp.int32),
        mesh=mesh,
        compiler_params=pltpu.CompilerParams(needs_layout_passes=False),
        scratch_types=[
            pltpu.VMEM((H,), jnp.int32),
            pltpu.VMEM((16 * _LANES * _LANES,), jnp.int32),
            pltpu.VMEM((_LANES,), jnp.int32),
        ],
    )(xi)


def _apply_body(thr_ref, x_ref, sparse_ref, mask_ref):
    x = x_ref[...]
    u = lax.bitcast_convert_type(jnp.abs(x), jnp.int32)
    keep = u >= thr_ref[...]
    mask_ref[...] = keep
    sparse_ref[...] = jnp.where(keep, x, 0.0)


def _tc_apply(x, thr, rows):
    B, H = x.shape
    return pl.pallas_call(
        _apply_body,
        grid=(B // rows,),
        in_specs=[
            pl.BlockSpec((rows, 1), lambda i: (i, 0)),
            pl.BlockSpec((rows, H), lambda i: (i, 0)),
        ],
        out_specs=[
            pl.BlockSpec((rows, H), lambda i: (i, 0)),
            pl.BlockSpec((rows, H), lambda i: (i, 0)),
        ],
        out_shape=[
            jax.ShapeDtypeStruct((B, H), jnp.float32),
            jax.ShapeDtypeStruct((B, H), jnp.bool_),
        ],
    )(thr, x)


def _select_body(x_ref, sparse_ref, mask_ref, *, k):
    x = x_ref[...]
    u = lax.bitcast_convert_type(jnp.abs(x), jnp.int32)

    def step(i, p):
        cand = p | (jnp.int32(1) << (30 - i))
        cnt = jnp.sum((u >= cand).astype(jnp.int32), axis=1, keepdims=True)
        return jnp.where(cnt >= k, cand, p)

    p0 = jnp.zeros((x.shape[0], 1), jnp.int32)
    thr = lax.fori_loop(0, 31, step, p0)
    keep = u >= thr
    mask_ref[...] = keep
    sparse_ref[...] = jnp.where(keep, x, 0.0)


def _tc_only(flat, k):
    B, H = flat.shape
    rows = 8 if B % 8 == 0 else 1
    return pl.pallas_call(
        functools.partial(_select_body, k=k),
        grid=(B // rows,),
        in_specs=[pl.BlockSpec((rows, H), lambda i: (i, 0))],
        out_specs=[
            pl.BlockSpec((rows, H), lambda i: (i, 0)),
            pl.BlockSpec((rows, H), lambda i: (i, 0)),
        ],
        out_shape=[
            jax.ShapeDtypeStruct((B, H), jnp.float32),
            jax.ShapeDtypeStruct((B, H), jnp.bool_),
        ],
    )(flat)


def kernel(x):
    flat = x if x.ndim == 2 else x.reshape(x.shape[0], -1)
    B, H = flat.shape
    k = max(1, int(H * 10.0 / 100.0))
    if B == 2 * _NW and H % _LANES == 0 and B % 8 == 0:
        xi = lax.bitcast_convert_type(flat, jnp.int32)
        thr_tiles = _sc_select(xi, k)
        thr = thr_tiles[:, :2].reshape(B, 1)
        sparse, mask = _tc_apply(flat, thr, 8)
    else:
        sparse, mask = _tc_only(flat, k)
    return sparse.reshape(x.shape), mask.reshape(x.shape)


# speedup vs baseline: 3.0392x; 3.0392x over previous
import functools

import jax
import jax.numpy as jnp
from jax import lax
from jax.experimental import pallas as pl
from jax.experimental.pallas import tpu as pltpu
from jax.experimental.pallas import tpu_sc as plsc

_NC, _NS, _LANES = 2, 16, 16
_NW = _NC * _NS

_LVL_BITS = (8, 8, 8, 7)
_LVL_SHIFTS = (23, 15, 7, 0)


def _sc_select_body(k, nvec, x_hbm, thr_hbm, xbuf0, xbuf1, hist0, hist1,
                    tvec, sem0, sem1):
    lane = lax.iota(jnp.int32, _LANES)
    ones = jnp.full((_LANES,), 1, jnp.int32)
    zeros16 = jnp.zeros((_LANES,), jnp.int32)
    wid = lax.axis_index("c") * _NS + lax.axis_index("s")

    c0 = pltpu.async_copy(x_hbm.at[wid * 2], xbuf0, sem0)
    c1 = pltpu.async_copy(x_hbm.at[wid * 2 + 1], xbuf1, sem1)
    c0.wait()
    c1.wait()

    kk0 = jnp.int32(k)
    kk1 = jnp.int32(k)
    p0 = jnp.int32(0)
    p1 = jnp.int32(0)
    for lvl in range(4):
        shift = _LVL_SHIFTS[lvl]
        nbits = _LVL_BITS[lvl]
        nb = 1 << nbits

        @plsc.parallel_loop(0, nb, unroll=4)
        def _(b):
            hist0[pl.ds(b * _LANES, _LANES)] = zeros16
            hist1[pl.ds(b * _LANES, _LANES)] = zeros16

        pfx0, pfx1 = p0, p1

        @plsc.parallel_loop(0, nvec, unroll=8)
        def _(i, lvl=lvl, shift=shift, nbits=nbits, nb=nb):
            v0 = plsc.bitcast(xbuf0[pl.ds(i * _LANES, _LANES)], jnp.int32)
            v1 = plsc.bitcast(xbuf1[pl.ds(i * _LANES, _LANES)], jnp.int32)
            v0 = v0 & jnp.int32(0x7FFFFFFF)
            v1 = v1 & jnp.int32(0x7FFFFFFF)
            i0 = ((v0 >> shift) & jnp.int32(nb - 1)) * _LANES + lane
            i1 = ((v1 >> shift) & jnp.int32(nb - 1)) * _LANES + lane
            if lvl == 0:
                plsc.addupdate_scatter(hist0, [i0], ones)
                plsc.addupdate_scatter(hist1, [i1], ones)
            else:
                plsc.addupdate_scatter(hist0, [i0], ones,
                                       mask=(v0 >> (shift + nbits)) == pfx0)
                plsc.addupdate_scatter(hist1, [i1], ones,
                                       mask=(v1 >> (shift + nbits)) == pfx1)

        @plsc.parallel_loop(0, nb, carry=(zeros16, zeros16))
        def _(t, acc, nb=nb):
            a0, a1 = acc
            bb = nb - 1 - t
            a0 = a0 + hist0[pl.ds(bb * _LANES, _LANES)]
            a1 = a1 + hist1[pl.ds(bb * _LANES, _LANES)]
            hist0[pl.ds(bb * _LANES, _LANES)] = a0
            hist1[pl.ds(bb * _LANES, _LANES)] = a1
            return (a0, a1)

        def search(hist, kk):
            lo = jnp.int32(0)
            step = nb >> 1
            while step >= 1:
                cand = lo + jnp.int32(step)
                candc = jnp.minimum(cand, nb - 1)
                sv = jnp.sum(hist[pl.ds(candc * _LANES, _LANES)])
                take = (cand <= nb - 1) & (sv >= kk)
                lo = jnp.where(take, cand, lo)
                step >>= 1
            upc = jnp.minimum(lo + 1, nb - 1)
            sv1 = jnp.sum(hist[pl.ds(upc * _LANES, _LANES)])
            above = jnp.where(lo + 1 <= nb - 1, sv1, jnp.int32(0))
            return lo, kk - above

        lo0, kk0 = search(hist0, kk0)
        lo1, kk1 = search(hist1, kk1)
        p0 = (p0 << nbits) | lo0
        p1 = (p1 << nbits) | lo1
    acc_thr = jnp.where(lane == 0, p0, jnp.where(lane == 1, p1, zeros16))
    tvec[...] = acc_thr
    pltpu.sync_copy(tvec, thr_hbm.at[wid])


def _sc_select(x, k):
    B, H = x.shape
    mesh = plsc.VectorSubcoreMesh(
        core_axis_name="c", subcore_axis_name="s", num_cores=_NC,
        num_subcores=_NS)
    body = functools.partial(_sc_select_body, k, H // _LANES)
    nbmax = 1 << max(_LVL_BITS)
    return pl.kernel(
        body,
        out_type=jax.ShapeDtypeStruct((_NW, _LANES), jnp.int32),
        mesh=mesh,
        compiler_params=pltpu.CompilerParams(needs_layout_passes=False),
        scratch_types=[
            pltpu.VMEM((H,), jnp.float32),
            pltpu.VMEM((H,), jnp.float32),
            pltpu.VMEM((nbmax * _LANES,), jnp.int32),
            pltpu.VMEM((nbmax * _LANES,), jnp.int32),
            pltpu.VMEM((_LANES,), jnp.int32),
            pltpu.SemaphoreType.DMA,
            pltpu.SemaphoreType.DMA,
        ],
    )(x)


def _apply_body(thr_ref, x_ref, sparse_ref, mask_ref):
    x = x_ref[...]
    u = lax.bitcast_convert_type(jnp.abs(x), jnp.int32)
    keep = u >= thr_ref[...]
    mask_ref[...] = keep
    sparse_ref[...] = jnp.where(keep, x, 0.0)


def _tc_apply(x, thr, rows):
    B, H = x.shape
    return pl.pallas_call(
        _apply_body,
        grid=(B // rows,),
        in_specs=[
            pl.BlockSpec((rows, 1), lambda i: (i, 0)),
            pl.BlockSpec((rows, H), lambda i: (i, 0)),
        ],
        out_specs=[
            pl.BlockSpec((rows, H), lambda i: (i, 0)),
            pl.BlockSpec((rows, H), lambda i: (i, 0)),
        ],
        out_shape=[
            jax.ShapeDtypeStruct((B, H), jnp.float32),
            jax.ShapeDtypeStruct((B, H), jnp.bool_),
        ],
    )(thr, x)


def _select_body(x_ref, sparse_ref, mask_ref, *, k):
    x = x_ref[...]
    u = lax.bitcast_convert_type(jnp.abs(x), jnp.int32)

    def step(i, p):
        cand = p | (jnp.int32(1) << (30 - i))
        cnt = jnp.sum((u >= cand).astype(jnp.int32), axis=1, keepdims=True)
        return jnp.where(cnt >= k, cand, p)

    p0 = jnp.zeros((x.shape[0], 1), jnp.int32)
    thr = lax.fori_loop(0, 31, step, p0)
    keep = u >= thr
    mask_ref[...] = keep
    sparse_ref[...] = jnp.where(keep, x, 0.0)


def _tc_only(flat, k):
    B, H = flat.shape
    rows = 8 if B % 8 == 0 else 1
    return pl.pallas_call(
        functools.partial(_select_body, k=k),
        grid=(B // rows,),
        in_specs=[pl.BlockSpec((rows, H), lambda i: (i, 0))],
        out_specs=[
            pl.BlockSpec((rows, H), lambda i: (i, 0)),
            pl.BlockSpec((rows, H), lambda i: (i, 0)),
        ],
        out_shape=[
            jax.ShapeDtypeStruct((B, H), jnp.float32),
            jax.ShapeDtypeStruct((B, H), jnp.bool_),
        ],
    )(flat)


def kernel(x):
    flat = x if x.ndim == 2 else x.reshape(x.shape[0], -1)
    B, H = flat.shape
    k = max(1, int(H * 10.0 / 100.0))
    if B == 2 * _NW and H % _LANES == 0 and B % 8 == 0:
        thr_tiles = _sc_select(flat, k)
        thr = thr_tiles[:, :2].reshape(B, 1)
        sparse, mask = _tc_apply(flat, thr, 8)
    else:
        sparse, mask = _tc_only(flat, k)
    return sparse.reshape(x.shape), mask.reshape(x.shape)
